# Initial kernel scaffold; baseline (speedup 1.0000x reference)
import functools

import jax
import jax.numpy as jnp
from jax import lax
from jax.experimental import pallas as pl
from jax.experimental.pallas import tpu as pltpu
from jax.experimental.pallas import tpu_sc as plsc

B = 16384
L = 200
D = 32
NUM_CLASSES = 10

_info = plsc.get_sparse_core_info()
NC, NS = _info.num_cores, _info.num_subcores
NW = NC * NS
BPW = B // NW
R = 4
NCHUNK = BPW // R
IDXC = R * L


def _reduce_chunk(rows, outst, c):
    for r in range(R):
        def inner(j, accs, r=r):
            a0, a1, a2, a3, a4, a5, a6, a7 = accs
            p = r * L + 4 * j
            a0 = a0 + rows[p, pl.ds(0, 16)]
            a1 = a1 + rows[p, pl.ds(16, 16)]
            a2 = a2 + rows[p + 1, pl.ds(0, 16)]
            a3 = a3 + rows[p + 1, pl.ds(16, 16)]
            a4 = a4 + rows[p + 2, pl.ds(0, 16)]
            a5 = a5 + rows[p + 2, pl.ds(16, 16)]
            a6 = a6 + rows[p + 3, pl.ds(0, 16)]
            a7 = a7 + rows[p + 3, pl.ds(16, 16)]
            return (a0, a1, a2, a3, a4, a5, a6, a7)

        z = jnp.zeros((16,), jnp.float32)
        accs = lax.fori_loop(0, L // 4, inner, (z,) * 8)
        lo = (accs[0] + accs[2]) + (accs[4] + accs[6])
        hi = (accs[1] + accs[3]) + (accs[5] + accs[7])
        row = c * R + r
        outst[row, pl.ds(0, 16)] = lo
        outst[row, pl.ds(16, 16)] = hi


@functools.partial(
    pl.kernel,
    out_type=jax.ShapeDtypeStruct((B, D), jnp.float32),
    mesh=plsc.VectorSubcoreMesh(core_axis_name="c", subcore_axis_name="s"),
    scratch_types=[
        pltpu.VMEM((IDXC,), jnp.int32),
        pltpu.VMEM((IDXC,), jnp.int32),
        pltpu.VMEM((IDXC, D), jnp.float32),
        pltpu.VMEM((IDXC, D), jnp.float32),
        pltpu.VMEM((BPW, D), jnp.float32),
        pltpu.SemaphoreType.DMA,
        pltpu.SemaphoreType.DMA,
    ],
)
def _sc_pool(x_hbm, emb_hbm, out_hbm, idx_a, idx_b, rows_a, rows_b, outst,
             sem_a, sem_b):
    wid = lax.axis_index("s") * NC + lax.axis_index("c")
    base = wid * BPW
    flat0 = base * L

    pltpu.sync_copy(x_hbm.at[pl.ds(flat0, IDXC)], idx_a)
    pltpu.async_copy(emb_hbm.at[idx_a], rows_a, sem_a)

    def body2(i, carry):
        c0 = 2 * i
        pltpu.sync_copy(x_hbm.at[pl.ds(flat0 + (c0 + 1) * IDXC, IDXC)], idx_b)
        pltpu.async_copy(emb_hbm.at[idx_b], rows_b, sem_b)
        pltpu.make_async_copy(emb_hbm.at[idx_a], rows_a, sem_a).wait()
        _reduce_chunk(rows_a, outst, c0)
        @pl.when(i < NCHUNK // 2 - 1)
        def _():
            pltpu.sync_copy(
                x_hbm.at[pl.ds(flat0 + (c0 + 2) * IDXC, IDXC)], idx_a)
            pltpu.async_copy(emb_hbm.at[idx_a], rows_a, sem_a)
        pltpu.make_async_copy(emb_hbm.at[idx_b], rows_b, sem_b).wait()
        _reduce_chunk(rows_b, outst, c0 + 1)
        return carry

    lax.fori_loop(0, NCHUNK // 2, body2, 0)
    pltpu.sync_copy(outst, out_hbm.at[pl.ds(base, BPW)])


def _tc_head(p_ref, wt_ref, b_ref, o_ref):
    p = p_ref[...] * (1.0 / L)
    z = lax.dot_general(p, wt_ref[...], (((1,), (0,)), ((), ())),
                        preferred_element_type=jnp.float32)
    z = z + b_ref[...]
    z = z - jnp.max(z, axis=1, keepdims=True)
    e = jnp.exp(z)
    o_ref[...] = e / jnp.sum(e, axis=1, keepdims=True)


def kernel(x, emb, W, b):
    pooled_sum = _sc_pool(x.reshape(-1), emb)
    blk = 2048
    y = pl.pallas_call(
        _tc_head,
        grid=(B // blk,),
        in_specs=[
            pl.BlockSpec((blk, D), lambda i: (i, 0)),
            pl.BlockSpec((D, NUM_CLASSES), lambda i: (0, 0)),
            pl.BlockSpec((1, NUM_CLASSES), lambda i: (0, 0)),
        ],
        out_specs=pl.BlockSpec((blk, NUM_CLASSES), lambda i: (i, 0)),
        out_shape=jax.ShapeDtypeStruct((B, NUM_CLASSES), jnp.float32),
    )(pooled_sum, W.T, b.reshape(1, NUM_CLASSES))
    return y


# speedup vs baseline: 15.3984x; 15.3984x over previous
import functools

import jax
import jax.numpy as jnp
from jax import lax
from jax.experimental import pallas as pl
from jax.experimental.pallas import tpu as pltpu
from jax.experimental.pallas import tpu_sc as plsc

B = 16384
L = 200
D = 32
NUM_CLASSES = 10

_info = plsc.get_sparse_core_info()
NC, NS = _info.num_cores, _info.num_subcores
NW = NC * NS
BPW = B // NW
R = 4
NCHUNK = BPW // R
IDXC = R * L


def _reduce_chunk(rows, outst, c):
    for r in range(R):
        def inner(j, accs, r=r):
            a0, a1, a2, a3, a4, a5, a6, a7 = accs
            p = r * L + 4 * j
            a0 = a0 + rows[p, pl.ds(0, 16)]
            a1 = a1 + rows[p, pl.ds(16, 16)]
            a2 = a2 + rows[p + 1, pl.ds(0, 16)]
            a3 = a3 + rows[p + 1, pl.ds(16, 16)]
            a4 = a4 + rows[p + 2, pl.ds(0, 16)]
            a5 = a5 + rows[p + 2, pl.ds(16, 16)]
            a6 = a6 + rows[p + 3, pl.ds(0, 16)]
            a7 = a7 + rows[p + 3, pl.ds(16, 16)]
            return (a0, a1, a2, a3, a4, a5, a6, a7)

        z = jnp.zeros((16,), jnp.float32)
        accs = lax.fori_loop(0, L // 4, inner, (z,) * 8)
        lo = (accs[0] + accs[2]) + (accs[4] + accs[6])
        hi = (accs[1] + accs[3]) + (accs[5] + accs[7])
        row = c * R + r
        outst[row, pl.ds(0, 16)] = lo
        outst[row, pl.ds(16, 16)] = hi


@functools.partial(
    pl.kernel,
    out_type=jax.ShapeDtypeStruct((B, D), jnp.float32),
    mesh=plsc.VectorSubcoreMesh(core_axis_name="c", subcore_axis_name="s"),
    compiler_params=pltpu.CompilerParams(use_tc_tiling_on_sc=False),
    scratch_types=[
        pltpu.VMEM((IDXC,), jnp.int32),
        pltpu.VMEM((IDXC,), jnp.int32),
        pltpu.VMEM((IDXC, D), jnp.float32),
        pltpu.VMEM((IDXC, D), jnp.float32),
        pltpu.VMEM((BPW, D), jnp.float32),
        pltpu.SemaphoreType.DMA,
        pltpu.SemaphoreType.DMA,
    ],
)
def _sc_pool(x_hbm, emb_hbm, out_hbm, idx_a, idx_b, rows_a, rows_b, outst,
             sem_a, sem_b):
    wid = lax.axis_index("s") * NC + lax.axis_index("c")
    base = wid * BPW
    flat0 = base * L

    pltpu.sync_copy(x_hbm.at[pl.ds(flat0, IDXC)], idx_a)
    pltpu.async_copy(emb_hbm.at[idx_a], rows_a, sem_a)

    def body2(i, carry):
        c0 = 2 * i
        pltpu.sync_copy(x_hbm.at[pl.ds(flat0 + (c0 + 1) * IDXC, IDXC)], idx_b)
        pltpu.async_copy(emb_hbm.at[idx_b], rows_b, sem_b)
        pltpu.make_async_copy(emb_hbm.at[idx_a], rows_a, sem_a).wait()
        _reduce_chunk(rows_a, outst, c0)
        @pl.when(i < NCHUNK // 2 - 1)
        def _():
            pltpu.sync_copy(
                x_hbm.at[pl.ds(flat0 + (c0 + 2) * IDXC, IDXC)], idx_a)
            pltpu.async_copy(emb_hbm.at[idx_a], rows_a, sem_a)
        pltpu.make_async_copy(emb_hbm.at[idx_b], rows_b, sem_b).wait()
        _reduce_chunk(rows_b, outst, c0 + 1)
        return carry

    lax.fori_loop(0, NCHUNK // 2, body2, 0)
    pltpu.sync_copy(outst, out_hbm.at[pl.ds(base, BPW)])


def _tc_head(p_ref, wt_ref, b_ref, o_ref):
    p = p_ref[...] * (1.0 / L)
    z = lax.dot_general(p, wt_ref[...], (((1,), (0,)), ((), ())),
                        preferred_element_type=jnp.float32)
    z = z + b_ref[...]
    z = z - jnp.max(z, axis=1, keepdims=True)
    e = jnp.exp(z)
    o_ref[...] = e / jnp.sum(e, axis=1, keepdims=True)


def kernel(x, emb, W, b):
    pooled_sum = _sc_pool(x.reshape(-1), emb)
    blk = 2048
    y = pl.pallas_call(
        _tc_head,
        grid=(B // blk,),
        in_specs=[
            pl.BlockSpec((blk, D), lambda i: (i, 0)),
            pl.BlockSpec((D, NUM_CLASSES), lambda i: (0, 0)),
            pl.BlockSpec((1, NUM_CLASSES), lambda i: (0, 0)),
        ],
        out_specs=pl.BlockSpec((blk, NUM_CLASSES), lambda i: (i, 0)),
        out_shape=jax.ShapeDtypeStruct((B, NUM_CLASSES), jnp.float32),
    )(pooled_sum, W.T, b.reshape(1, NUM_CLASSES))
    return y
